# baseline (speedup 1.0000x reference)
import functools

import jax
import jax.numpy as jnp
from jax import lax
from jax.experimental import pallas as pl
from jax.experimental.pallas import tpu as pltpu
from jax.experimental.pallas import tpu_sc as plsc

NC = 2
NS = 16
NW = NC * NS

BATCH = 4096
MAX_LEN = 200
DEPTH = 32

PT = MAX_LEN // 8
TT = BATCH // 128
AT = DEPTH // 8

TG = 4
NGRP = MAX_LEN * (TT // TG)
GPW = NGRP // NW
GROUP_ROWS = TG * 128

_mesh = plsc.VectorSubcoreMesh(core_axis_name="c", subcore_axis_name="s")


@functools.partial(
    pl.kernel,
    out_type=jax.ShapeDtypeStruct((MAX_LEN, AT, TT, 8, 128), jnp.float32),
    mesh=_mesh,
    compiler_params=pltpu.CompilerParams(
        use_tc_tiling_on_sc=False, needs_layout_passes=False
    ),
    scratch_types=[
        pltpu.VMEM((2, TG, 128), jnp.int32),
        pltpu.VMEM((2, GROUP_ROWS, DEPTH), jnp.float32),
        pltpu.VMEM((2, AT, TG, 8, 128), jnp.float32),
        pltpu.VMEM((MAX_LEN, DEPTH), jnp.float32),
        pltpu.SemaphoreType.DMA,
        pltpu.SemaphoreType.DMA,
        pltpu.SemaphoreType.DMA,
    ],
)
def _embed(seq_hbm, tok_hbm, pos_hbm, out_hbm,
           idx_v, rows_v, trans_v, pos_v, isem, gsem, wsem):
    wid = lax.axis_index("s") * NC + lax.axis_index("c")
    g0 = wid * GPW

    pltpu.sync_copy(pos_hbm, pos_v)
    iota16 = lax.iota(jnp.int32, 16)

    def coords(g):
        l = g // (TT // TG)
        tg = g % (TT // TG)
        return l, l // 8, l % 8, tg * TG

    def stage_ids(g, b):
        _, p, s, t0 = coords(g)
        return [
            pltpu.async_copy(seq_hbm.at[p, t0 + tl, s], idx_v.at[b, tl], isem)
            for tl in range(TG)
        ]

    def launch_gathers(b):
        for tl in range(TG):
            pltpu.async_copy(
                tok_hbm.at[idx_v.at[b, tl]],
                rows_v.at[b].at[pl.ds(tl * 128, 128)],
                gsem,
            )

    def wait_gathers(b):
        pltpu.make_async_copy(
            tok_hbm.at[pl.ds(0, GROUP_ROWS)], rows_v.at[b], gsem
        ).wait()

    def transpose_add(g, b):
        l, _, _, _ = coords(g)
        lvec = jnp.full((16,), l, dtype=jnp.int32)

        def d_body(d, carry):
            a = d // 8
            dsub = d % 8
            col = jnp.full((16,), d, dtype=jnp.int32)
            ps = plsc.load_gather(pos_v, [lvec, col])
            for tl in range(TG):
                for uc in range(8):
                    row = tl * 128 + uc * 16
                    vals = plsc.load_gather(rows_v.at[b], [iota16 + row, col])
                    trans_v[b, a, tl, dsub, pl.ds(uc * 16, 16)] = vals + ps
            return carry

        lax.fori_loop(0, DEPTH, d_body, 0)

    def writeback(g, b):
        l, _, _, t0 = coords(g)
        return [
            pltpu.async_copy(
                trans_v.at[b, a], out_hbm.at[l, a, pl.ds(t0, TG)], wsem
            )
            for a in range(AT)
        ]

    for b in (0, 1):
        for c in stage_ids(g0 + b, b):
            c.wait()
        launch_gathers(b)

    def pair_body(i, carry):
        for b in (0, 1):
            ci = 2 * i + b
            g = g0 + ci
            wait_gathers(b)
            transpose_add(g, b)
            wbs = writeback(g, b)

            @pl.when(ci + 2 < GPW)
            def _next():
                ids = stage_ids(g + 2, b)
                for c in wbs:
                    c.wait()
                for c in ids:
                    c.wait()
                launch_gathers(b)

            @pl.when(ci + 2 >= GPW)
            def _tail():
                for c in wbs:
                    c.wait()

        return carry

    lax.fori_loop(0, GPW // 2, pair_body, 0)


def kernel(seq, token_table, pos_table):
    seq4d = seq.T.reshape(PT, 8, TT, 128).transpose(0, 2, 1, 3)
    out5d = _embed(seq4d, token_table, pos_table)
    return out5d.transpose(2, 4, 0, 1, 3).reshape(BATCH, MAX_LEN, DEPTH)


# speedup vs baseline: 1.5402x; 1.5402x over previous
import functools

import jax
import jax.numpy as jnp
from jax import lax
from jax.experimental import pallas as pl
from jax.experimental.pallas import tpu as pltpu
from jax.experimental.pallas import tpu_sc as plsc

NC = 2
NS = 16
NW = NC * NS

BATCH = 4096
MAX_LEN = 200
DEPTH = 32

PT = MAX_LEN // 8
TT = BATCH // 128
AT = DEPTH // 8

TG = 4
NGRP = MAX_LEN * (TT // TG)
GPW = NGRP // NW
GROUP_ROWS = TG * 128
TRANS = AT * TG * 8 * 128

OUT_FLAT = MAX_LEN * AT * TT * 8 * 128

_mesh = plsc.VectorSubcoreMesh(core_axis_name="c", subcore_axis_name="s")


@functools.partial(
    pl.kernel,
    out_type=jax.ShapeDtypeStruct((OUT_FLAT,), jnp.float32),
    mesh=_mesh,
    compiler_params=pltpu.CompilerParams(
        use_tc_tiling_on_sc=False, needs_layout_passes=False
    ),
    scratch_types=[
        pltpu.VMEM((2, TG, 128), jnp.int32),
        pltpu.VMEM((2, GROUP_ROWS, DEPTH), jnp.float32),
        pltpu.VMEM((2, TRANS), jnp.float32),
        pltpu.VMEM((MAX_LEN * DEPTH,), jnp.float32),
        pltpu.VMEM((32 * 16,), jnp.int32),
        pltpu.VMEM((32 * 16,), jnp.int32),
        pltpu.SemaphoreType.DMA,
        pltpu.SemaphoreType.DMA,
        pltpu.SemaphoreType.DMA,
    ],
)
def _embed(seq_hbm, tok_hbm, pos_hbm, out_hbm,
           idx_v, rows_v, trans_v, pos_v, perm_v, skew_v,
           isem, gsem, wsem):
    wid = lax.axis_index("s") * NC + lax.axis_index("c")
    g0 = wid * GPW

    pltpu.sync_copy(pos_hbm, pos_v)
    iota16 = lax.iota(jnp.int32, 16)

    for kc in range(32):
        k = kc & 15
        c0 = (kc >> 4) * 16
        dvec = c0 + ((iota16 + k) & 15)
        fvec = ((dvec >> 3) << 12) + ((dvec & 7) << 7)
        perm_v[pl.ds(kc * 16, 16)] = dvec
        skew_v[pl.ds(kc * 16, 16)] = fvec + iota16

    def coords(g):
        l = g // (TT // TG)
        tg = g % (TT // TG)
        return l, l // 8, l % 8, tg * TG

    def stage_ids(g, b):
        _, p, s, t0 = coords(g)
        return [
            pltpu.async_copy(seq_hbm.at[p, t0 + tl, s], idx_v.at[b, tl], isem)
            for tl in range(TG)
        ]

    def launch_gathers(b):
        for tl in range(TG):
            pltpu.async_copy(
                tok_hbm.at[idx_v.at[b, tl]],
                rows_v.at[b].at[pl.ds(tl * 128, 128)],
                gsem,
            )

    def wait_gathers(b):
        pltpu.make_async_copy(
            tok_hbm.at[pl.ds(0, GROUP_ROWS)], rows_v.at[b], gsem
        ).wait()

    def transpose_add(g, b):
        l = g // (TT // TG)
        lbase = l * DEPTH

        def kc_body(kc, carry):
            dvec = perm_v[pl.ds(kc * 16, 16)]
            skew = skew_v[pl.ds(kc * 16, 16)]
            ps = plsc.load_gather(pos_v, [lbase + dvec])
            for tl in range(TG):
                for rb in range(8):
                    r0 = tl * 128 + rb * 16
                    vals = plsc.load_gather(rows_v.at[b], [iota16 + r0, dvec])
                    plsc.store_scatter(
                        trans_v.at[b],
                        [skew + (tl * 1024 + rb * 16)],
                        vals + ps,
                    )
            return carry

        lax.fori_loop(0, 32, kc_body, 0)

    def writeback(g, b):
        l, _, _, t0 = coords(g)
        base = pl.multiple_of(l * (AT * TT * 1024) + t0 * 1024, 1024)
        return [
            pltpu.async_copy(
                trans_v.at[b].at[pl.ds(a * 4096, 4096)],
                out_hbm.at[pl.ds(base + a * (TT * 1024), 4096)],
                wsem,
            )
            for a in range(AT)
        ]

    for b in (0, 1):
        for c in stage_ids(g0 + b, b):
            c.wait()
        launch_gathers(b)

    def pair_body(i, carry):
        for b in (0, 1):
            ci = 2 * i + b
            g = g0 + ci
            wait_gathers(b)
            transpose_add(g, b)
            wbs = writeback(g, b)

            @pl.when(ci + 2 < GPW)
            def _next():
                ids = stage_ids(g + 2, b)
                for c in wbs:
                    c.wait()
                for c in ids:
                    c.wait()
                launch_gathers(b)

            @pl.when(ci + 2 >= GPW)
            def _tail():
                for c in wbs:
                    c.wait()

        return carry

    lax.fori_loop(0, GPW // 2, pair_body, 0)


def kernel(seq, token_table, pos_table):
    seq4d = seq.T.reshape(PT, 8, TT, 128).transpose(0, 2, 1, 3)
    out_flat = _embed(seq4d, token_table, pos_table.reshape(-1))
    out5d = out_flat.reshape(MAX_LEN, AT, TT, 8, 128)
    return out5d.transpose(2, 4, 0, 1, 3).reshape(BATCH, MAX_LEN, DEPTH)


# speedup vs baseline: 1.5479x; 1.0050x over previous
import functools

import jax
import jax.numpy as jnp
from jax import lax
from jax.experimental import pallas as pl
from jax.experimental.pallas import tpu as pltpu
from jax.experimental.pallas import tpu_sc as plsc

NC = 2
NS = 16
NW = NC * NS

BATCH = 4096
MAX_LEN = 200
DEPTH = 32

PT = MAX_LEN // 8
TT = BATCH // 128
AT = DEPTH // 8

TG = 4
NGRP = MAX_LEN * (TT // TG)
GPW = NGRP // NW
GROUP_ROWS = TG * 128
TRANS = AT * TG * 8 * 128

OUT_FLAT = MAX_LEN * AT * TT * 8 * 128

_mesh = plsc.VectorSubcoreMesh(core_axis_name="c", subcore_axis_name="s")


@functools.partial(
    pl.kernel,
    out_type=jax.ShapeDtypeStruct((OUT_FLAT,), jnp.float32),
    mesh=_mesh,
    compiler_params=pltpu.CompilerParams(
        use_tc_tiling_on_sc=False, needs_layout_passes=False
    ),
    scratch_types=[
        pltpu.VMEM((2, TG, 128), jnp.int32),
        pltpu.VMEM((2, GROUP_ROWS, DEPTH), jnp.float32),
        pltpu.VMEM((2, TRANS), jnp.float32),
        pltpu.VMEM((MAX_LEN * DEPTH,), jnp.float32),
        pltpu.VMEM((32 * 16,), jnp.int32),
        pltpu.VMEM((32 * 16,), jnp.int32),
        pltpu.SemaphoreType.DMA,
        pltpu.SemaphoreType.DMA,
        pltpu.SemaphoreType.DMA,
    ],
)
def _embed(seq_hbm, tok_hbm, pos_hbm, out_hbm,
           idx_v, rows_v, trans_v, pos_v, perm_v, skew_v,
           isem, gsem, wsem):
    wid = lax.axis_index("s") * NC + lax.axis_index("c")
    g0 = wid * GPW

    pltpu.sync_copy(pos_hbm, pos_v)
    iota16 = lax.iota(jnp.int32, 16)

    for kc in range(32):
        k = kc & 15
        c0 = (kc >> 4) * 16
        dvec = c0 + ((iota16 + k) & 15)
        fvec = ((dvec >> 3) << 12) + ((dvec & 7) << 7)
        perm_v[pl.ds(kc * 16, 16)] = dvec
        skew_v[pl.ds(kc * 16, 16)] = fvec + iota16

    def coords(g):
        l = g // (TT // TG)
        tg = g % (TT // TG)
        return l, l // 8, l % 8, tg * TG

    def stage_ids(g, b):
        _, p, s, t0 = coords(g)
        return [
            pltpu.async_copy(seq_hbm.at[p, pl.ds(t0, TG), s], idx_v.at[b], isem)
        ]

    def launch_gathers(b):
        for tl in range(TG):
            pltpu.async_copy(
                tok_hbm.at[idx_v.at[b, tl]],
                rows_v.at[b].at[pl.ds(tl * 128, 128)],
                gsem,
            )

    def wait_gathers(b):
        pltpu.make_async_copy(
            tok_hbm.at[pl.ds(0, GROUP_ROWS)], rows_v.at[b], gsem
        ).wait()

    def transpose_add(g, b):
        l = g // (TT // TG)
        lbase = l * DEPTH

        def kc_body(kc, carry):
            dvec = perm_v[pl.ds(kc * 16, 16)]
            skew = skew_v[pl.ds(kc * 16, 16)]
            ps = plsc.load_gather(pos_v, [lbase + dvec])
            for tl in range(TG):
                for rb in range(8):
                    r0 = tl * 128 + rb * 16
                    vals = plsc.load_gather(rows_v.at[b], [iota16 + r0, dvec])
                    plsc.store_scatter(
                        trans_v.at[b],
                        [skew + (tl * 1024 + rb * 16)],
                        vals + ps,
                    )
            return carry

        lax.fori_loop(0, 32, kc_body, 0)

    def writeback(g, b):
        l, _, _, t0 = coords(g)
        base = pl.multiple_of(l * (AT * TT * 1024) + t0 * 1024, 1024)
        return [
            pltpu.async_copy(
                trans_v.at[b].at[pl.ds(a * 4096, 4096)],
                out_hbm.at[pl.ds(base + a * (TT * 1024), 4096)],
                wsem,
            )
            for a in range(AT)
        ]

    for b in (0, 1):
        for c in stage_ids(g0 + b, b):
            c.wait()
        launch_gathers(b)

    def wait_writeback(b):
        pltpu.make_async_copy(
            out_hbm.at[pl.ds(0, TRANS)], trans_v.at[b], wsem
        ).wait()

    def pair_body(i, carry):
        for b in (0, 1):
            ci = 2 * i + b
            g = g0 + ci
            wait_gathers(b)

            @pl.when(ci >= 2)
            def _reclaim():
                wait_writeback(b)

            transpose_add(g, b)
            writeback(g, b)

            @pl.when(ci + 2 < GPW)
            def _next():
                ids = stage_ids(g + 2, b)
                for c in ids:
                    c.wait()
                launch_gathers(b)

        return carry

    lax.fori_loop(0, GPW // 2, pair_body, 0)
    wait_writeback(0)
    wait_writeback(1)


def kernel(seq, token_table, pos_table):
    seq4d = seq.T.reshape(PT, 8, TT, 128).transpose(0, 2, 1, 3)
    out_flat = _embed(seq4d, token_table, pos_table.reshape(-1))
    out5d = out_flat.reshape(MAX_LEN, AT, TT, 8, 128)
    return out5d.transpose(2, 4, 0, 1, 3).reshape(BATCH, MAX_LEN, DEPTH)


# speedup vs baseline: 1.8952x; 1.2243x over previous
import functools

import jax
import jax.numpy as jnp
from jax import lax
from jax.experimental import pallas as pl
from jax.experimental.pallas import tpu as pltpu
from jax.experimental.pallas import tpu_sc as plsc

NC = 2
NS = 16
NW = NC * NS

BATCH = 4096
MAX_LEN = 200
DEPTH = 32
VOCAB_ROWS = 1000000

PT = MAX_LEN // 8
TT = BATCH // 128
AT = DEPTH // 8

TG = 4
NGRP = MAX_LEN * (TT // TG)
GPW = NGRP // NW
GROUP_ROWS = TG * 128
TRANS = AT * TG * 8 * 128

OUT_FLAT = MAX_LEN * AT * TT * 8 * 128

_mesh = plsc.VectorSubcoreMesh(core_axis_name="c", subcore_axis_name="s")


@functools.partial(
    pl.kernel,
    out_type=jax.ShapeDtypeStruct((OUT_FLAT,), jnp.float32),
    mesh=_mesh,
    compiler_params=pltpu.CompilerParams(
        use_tc_tiling_on_sc=False, needs_layout_passes=False
    ),
    scratch_types=[
        pltpu.VMEM((2, TG, 128), jnp.int32),
        pltpu.VMEM((2, GROUP_ROWS, DEPTH), jnp.float32),
        pltpu.VMEM((2, TRANS), jnp.float32),
        pltpu.VMEM((MAX_LEN * DEPTH,), jnp.float32),
        pltpu.VMEM((32 * 16,), jnp.int32),
        pltpu.VMEM((32 * 16,), jnp.int32),
        pltpu.SemaphoreType.DMA,
        pltpu.SemaphoreType.DMA,
        pltpu.SemaphoreType.DMA,
    ],
)
def _embed(seq_hbm, tok_hbm, pos_hbm, out_hbm,
           idx_v, rows_v, trans_v, pos_v, perm_v, skew_v,
           isem, gsem, wsem):
    wid = lax.axis_index("s") * NC + lax.axis_index("c")
    g0 = wid * GPW

    pltpu.sync_copy(pos_hbm, pos_v)
    iota16 = lax.iota(jnp.int32, 16)

    for kc in range(32):
        k = kc & 15
        c0 = (kc >> 4) * 16
        dvec = c0 + ((iota16 + k) & 15)
        fvec = ((dvec >> 3) << 12) + ((dvec & 7) << 7)
        perm_v[pl.ds(kc * 16, 16)] = dvec
        skew_v[pl.ds(kc * 16, 16)] = fvec + iota16

    def coords(g):
        l = g // (TT // TG)
        tg = g % (TT // TG)
        return l, l // 8, l % 8, tg * TG

    def stage_ids(g, b):
        _, p, s, t0 = coords(g)
        return [
            pltpu.async_copy(seq_hbm.at[p, pl.ds(t0, TG), s], idx_v.at[b], isem)
        ]

    def launch_gathers(b):
        for tl in range(TG):
            pltpu.async_copy(
                tok_hbm.at[idx_v.at[b, tl]],
                rows_v.at[b].at[pl.ds(tl * 128, 128)],
                gsem,
            )

    def wait_gathers(b):
        pltpu.make_async_copy(
            tok_hbm.at[pl.ds(0, GROUP_ROWS)], rows_v.at[b], gsem
        ).wait()

    def transpose_add(g, b):
        l = g // (TT // TG)
        lbase = l * DEPTH

        @plsc.parallel_loop(0, 32, unroll=2)
        def kc_body(kc):
            dvec = perm_v[pl.ds(kc * 16, 16)]
            skew = skew_v[pl.ds(kc * 16, 16)]
            ps = plsc.load_gather(pos_v, [lbase + dvec])
            for tl in range(TG):
                for rb in range(8):
                    r0 = tl * 128 + rb * 16
                    vals = plsc.load_gather(rows_v.at[b], [iota16 + r0, dvec])
                    plsc.store_scatter(
                        trans_v.at[b],
                        [skew + (tl * 1024 + rb * 16)],
                        vals + ps,
                    )

    def writeback(g, b):
        l, _, _, t0 = coords(g)
        base = pl.multiple_of(l * (AT * TT * 1024) + t0 * 1024, 1024)
        return [
            pltpu.async_copy(
                trans_v.at[b].at[pl.ds(a * 4096, 4096)],
                out_hbm.at[pl.ds(base + a * (TT * 1024), 4096)],
                wsem,
            )
            for a in range(AT)
        ]

    for b in (0, 1):
        for c in stage_ids(g0 + b, b):
            c.wait()
        launch_gathers(b)

    def wait_writeback(b):
        pltpu.make_async_copy(
            out_hbm.at[pl.ds(0, TRANS)], trans_v.at[b], wsem
        ).wait()

    def pair_body(i, carry):
        for b in (0, 1):
            ci = 2 * i + b
            g = g0 + ci
            wait_gathers(b)

            @pl.when(ci >= 2)
            def _reclaim():
                wait_writeback(b)

            transpose_add(g, b)
            writeback(g, b)

            @pl.when(ci + 2 < GPW)
            def _next():
                ids = stage_ids(g + 2, b)
                for c in ids:
                    c.wait()
                launch_gathers(b)

        return carry

    lax.fori_loop(0, GPW // 2, pair_body, 0)
    wait_writeback(0)
    wait_writeback(1)


def kernel(seq, token_table, pos_table):
    seq4d = seq.T.reshape(PT, 8, TT, 128).transpose(0, 2, 1, 3)
    tok_lin = jax.lax.optimization_barrier(
        token_table.reshape(VOCAB_ROWS // 4, 128)
    ).reshape(VOCAB_ROWS, DEPTH)
    out_flat = _embed(seq4d, tok_lin, pos_table.reshape(-1))
    out5d = out_flat.reshape(MAX_LEN, AT, TT, 8, 128)
    return out5d.transpose(2, 4, 0, 1, 3).reshape(BATCH, MAX_LEN, DEPTH)


# speedup vs baseline: 3.0975x; 1.6344x over previous
import functools

import jax
import jax.numpy as jnp
from jax import lax
from jax.experimental import pallas as pl
from jax.experimental.pallas import tpu as pltpu
from jax.experimental.pallas import tpu_sc as plsc

NC = 2
NS = 16
NW = NC * NS

BATCH = 4096
MAX_LEN = 200
DEPTH = 32
VOCAB_ROWS = 1000000

PT = MAX_LEN // 8
TT = BATCH // 128
AT = DEPTH // 8

TG = 4
NGRP = MAX_LEN * (TT // TG)
GPW = NGRP // NW
GROUP_ROWS = TG * 128
TRANS = AT * TG * 8 * 128

OUT_FLAT = MAX_LEN * AT * TT * 8 * 128

VB = 7813
VOCAB_PAD = VB * 128

_mesh = plsc.VectorSubcoreMesh(core_axis_name="c", subcore_axis_name="s")


@functools.partial(
    pl.kernel,
    out_type=jax.ShapeDtypeStruct((VOCAB_PAD * DEPTH,), jnp.float32),
    mesh=_mesh,
    compiler_params=pltpu.CompilerParams(
        use_tc_tiling_on_sc=False, needs_layout_passes=False
    ),
    scratch_types=[
        pltpu.VMEM((2, DEPTH, 128), jnp.float32),
        pltpu.VMEM((2, 128 * DEPTH), jnp.float32),
        pltpu.VMEM((32 * 16,), jnp.int32),
        pltpu.VMEM((32 * 16,), jnp.int32),
        pltpu.SemaphoreType.DMA,
        pltpu.SemaphoreType.DMA,
    ],
)
def _table_transpose(tiles_hbm, lin_hbm, tiles_v, rows_v, dtab_v, sttab_v,
                     tsem, wsem):
    wid = lax.axis_index("s") * NC + lax.axis_index("c")
    iota16 = lax.iota(jnp.int32, 16)

    for kc in range(32):
        k = kc & 15
        c0 = (kc >> 4) * 16
        dvec = c0 + ((iota16 + k) & 15)
        dtab_v[pl.ds(kc * 16, 16)] = dvec
        sttab_v[pl.ds(kc * 16, 16)] = iota16 * DEPTH + dvec

    nvb = 244 + jnp.where(wid < VB - 32 * 244, 1, 0)

    def stage(i, b):
        vb = wid + 32 * i
        for a in range(AT):
            pltpu.async_copy(
                tiles_hbm.at[a, vb], tiles_v.at[b].at[pl.ds(a * 8, 8)], tsem
            )

    def wait_tiles(b):
        for a in range(AT):
            pltpu.make_async_copy(
                tiles_hbm.at[0, 0], tiles_v.at[b].at[pl.ds(a * 8, 8)], tsem
            ).wait()

    def transpose(b):
        @plsc.parallel_loop(0, 32, unroll=2)
        def kc_body(kc):
            dvec = dtab_v[pl.ds(kc * 16, 16)]
            st = sttab_v[pl.ds(kc * 16, 16)]
            for ub in range(8):
                vals = plsc.load_gather(tiles_v.at[b], [dvec, iota16 + ub * 16])
                plsc.store_scatter(rows_v.at[b], [st + ub * 16 * DEPTH], vals)

    def writeback(i, b):
        vb = wid + 32 * i
        base = pl.multiple_of(vb * (128 * DEPTH), 128 * DEPTH)
        pltpu.async_copy(
            rows_v.at[b].at[pl.ds(0, 128 * DEPTH)],
            lin_hbm.at[pl.ds(base, 128 * DEPTH)],
            wsem,
        )

    def wait_writeback(b):
        pltpu.make_async_copy(
            lin_hbm.at[pl.ds(0, 128 * DEPTH)], rows_v.at[b], wsem
        ).wait()

    stage(0, 0)
    stage(1, 1)

    def pair_body(i, carry):
        for b in (0, 1):
            idx = 2 * i + b
            wait_tiles(b)

            @pl.when(idx >= 2)
            def _reclaim():
                wait_writeback(b)

            transpose(b)
            writeback(idx, b)

            @pl.when(idx + 2 < nvb)
            def _next():
                stage(idx + 2, b)

        return carry

    lax.fori_loop(0, 122, pair_body, 0)

    @pl.when(nvb > 244)
    def _tail():
        wait_tiles(0)
        wait_writeback(0)
        transpose(0)
        writeback(244, 0)

    wait_writeback(0)
    wait_writeback(1)


@functools.partial(
    pl.kernel,
    out_type=jax.ShapeDtypeStruct((OUT_FLAT,), jnp.float32),
    mesh=_mesh,
    compiler_params=pltpu.CompilerParams(
        use_tc_tiling_on_sc=False, needs_layout_passes=False
    ),
    scratch_types=[
        pltpu.VMEM((2, TG, 128), jnp.int32),
        pltpu.VMEM((2, GROUP_ROWS, DEPTH), jnp.float32),
        pltpu.VMEM((2, TRANS), jnp.float32),
        pltpu.VMEM((MAX_LEN * DEPTH,), jnp.float32),
        pltpu.VMEM((32 * 16,), jnp.int32),
        pltpu.VMEM((32 * 16,), jnp.int32),
        pltpu.SemaphoreType.DMA,
        pltpu.SemaphoreType.DMA,
        pltpu.SemaphoreType.DMA,
    ],
)
def _embed(seq_hbm, tok_hbm, pos_hbm, out_hbm,
           idx_v, rows_v, trans_v, pos_v, perm_v, skew_v,
           isem, gsem, wsem):
    wid = lax.axis_index("s") * NC + lax.axis_index("c")
    g0 = wid * GPW

    pltpu.sync_copy(pos_hbm, pos_v)
    iota16 = lax.iota(jnp.int32, 16)

    for kc in range(32):
        k = kc & 15
        c0 = (kc >> 4) * 16
        dvec = c0 + ((iota16 + k) & 15)
        fvec = ((dvec >> 3) << 12) + ((dvec & 7) << 7)
        perm_v[pl.ds(kc * 16, 16)] = dvec
        skew_v[pl.ds(kc * 16, 16)] = fvec + iota16

    def coords(g):
        l = g // (TT // TG)
        tg = g % (TT // TG)
        return l, l // 8, l % 8, tg * TG

    def stage_ids(g, b):
        _, p, s, t0 = coords(g)
        return [
            pltpu.async_copy(seq_hbm.at[p, pl.ds(t0, TG), s], idx_v.at[b], isem)
        ]

    def launch_gathers(b):
        for tl in range(TG):
            pltpu.async_copy(
                tok_hbm.at[idx_v.at[b, tl]],
                rows_v.at[b].at[pl.ds(tl * 128, 128)],
                gsem,
            )

    def wait_gathers(b):
        pltpu.make_async_copy(
            tok_hbm.at[pl.ds(0, GROUP_ROWS)], rows_v.at[b], gsem
        ).wait()

    def transpose_add(g, b):
        l = g // (TT // TG)
        lbase = l * DEPTH

        @plsc.parallel_loop(0, 32, unroll=2)
        def kc_body(kc):
            dvec = perm_v[pl.ds(kc * 16, 16)]
            skew = skew_v[pl.ds(kc * 16, 16)]
            ps = plsc.load_gather(pos_v, [lbase + dvec])
            for tl in range(TG):
                for rb in range(8):
                    r0 = tl * 128 + rb * 16
                    vals = plsc.load_gather(rows_v.at[b], [iota16 + r0, dvec])
                    plsc.store_scatter(
                        trans_v.at[b],
                        [skew + (tl * 1024 + rb * 16)],
                        vals + ps,
                    )

    def writeback(g, b):
        l, _, _, t0 = coords(g)
        base = pl.multiple_of(l * (AT * TT * 1024) + t0 * 1024, 1024)
        return [
            pltpu.async_copy(
                trans_v.at[b].at[pl.ds(a * 4096, 4096)],
                out_hbm.at[pl.ds(base + a * (TT * 1024), 4096)],
                wsem,
            )
            for a in range(AT)
        ]

    for b in (0, 1):
        for c in stage_ids(g0 + b, b):
            c.wait()
        launch_gathers(b)

    def wait_writeback(b):
        pltpu.make_async_copy(
            out_hbm.at[pl.ds(0, TRANS)], trans_v.at[b], wsem
        ).wait()

    def pair_body(i, carry):
        for b in (0, 1):
            ci = 2 * i + b
            g = g0 + ci
            wait_gathers(b)

            @pl.when(ci >= 2)
            def _reclaim():
                wait_writeback(b)

            transpose_add(g, b)
            writeback(g, b)

            @pl.when(ci + 2 < GPW)
            def _next():
                ids = stage_ids(g + 2, b)
                for c in ids:
                    c.wait()
                launch_gathers(b)

        return carry

    lax.fori_loop(0, GPW // 2, pair_body, 0)
    wait_writeback(0)
    wait_writeback(1)


def kernel(seq, token_table, pos_table):
    seq4d = seq.T.reshape(PT, 8, TT, 128).transpose(0, 2, 1, 3)
    tok_pad_t = jnp.pad(token_table.T, ((0, 0), (0, VOCAB_PAD - VOCAB_ROWS)))
    tok_tiles = tok_pad_t.reshape(AT, 8, VB, 128).transpose(0, 2, 1, 3)
    tok_lin = _table_transpose(tok_tiles).reshape(VOCAB_PAD, DEPTH)
    out_flat = _embed(seq4d, tok_lin, pos_table.reshape(-1))
    out5d = out_flat.reshape(MAX_LEN, AT, TT, 8, 128)
    return out5d.transpose(2, 4, 0, 1, 3).reshape(BATCH, MAX_LEN, DEPTH)


# speedup vs baseline: 3.8151x; 1.2317x over previous
import functools

import jax
import jax.numpy as jnp
from jax import lax
from jax.experimental import pallas as pl
from jax.experimental.pallas import tpu as pltpu
from jax.experimental.pallas import tpu_sc as plsc

NC = 2
NS = 16
NW = NC * NS

BATCH = 4096
MAX_LEN = 200
DEPTH = 32
VOCAB_ROWS = 1000000

PT = MAX_LEN // 8
TT = BATCH // 128
AT = DEPTH // 8

TG = 4
NGRP = MAX_LEN * (TT // TG)
GPW = NGRP // NW
GROUP_ROWS = TG * 128
TRANS = AT * TG * 8 * 128

OUT_FLAT = MAX_LEN * AT * TT * 8 * 128

VB_FULL = 7812
TAIL0 = VB_FULL * 128

_mesh = plsc.VectorSubcoreMesh(core_axis_name="c", subcore_axis_name="s")


@functools.partial(
    pl.kernel,
    out_type=jax.ShapeDtypeStruct((VOCAB_ROWS * DEPTH,), jnp.float32),
    mesh=_mesh,
    compiler_params=pltpu.CompilerParams(
        use_tc_tiling_on_sc=True, needs_layout_passes=False
    ),
    scratch_types=[
        pltpu.VMEM((DEPTH, 128), jnp.float32),
        pltpu.VMEM((DEPTH, 128), jnp.float32),
        pltpu.VMEM((128 * DEPTH,), jnp.float32),
        pltpu.VMEM((128 * DEPTH,), jnp.float32),
        pltpu.VMEM((32 * 16,), jnp.int32),
        pltpu.VMEM((32 * 16,), jnp.int32),
        pltpu.VMEM(((VOCAB_ROWS - TAIL0) * DEPTH,), jnp.float32),
        pltpu.SemaphoreType.DMA,
        pltpu.SemaphoreType.DMA,
    ],
)
def _table_transpose(tiles_hbm, tail_hbm, lin_hbm, tiles_v0, tiles_v1,
                     rows_v0, rows_v1, dtab_v, sttab_v, tail_v, tsem, wsem):
    tiles_bufs = (tiles_v0, tiles_v1)
    rows_bufs = (rows_v0, rows_v1)
    wid = lax.axis_index("s") * NC + lax.axis_index("c")
    iota16 = lax.iota(jnp.int32, 16)

    for kc in range(32):
        k = kc & 15
        c0 = (kc >> 4) * 16
        dvec = c0 + ((iota16 + k) & 15)
        dtab_v[pl.ds(kc * 16, 16)] = dvec
        sttab_v[pl.ds(kc * 16, 16)] = iota16 * DEPTH + dvec

    nvb = 244 + jnp.where(wid < VB_FULL - 32 * 244, 1, 0)

    @pl.when(wid == 0)
    def _tail_rows():
        pltpu.sync_copy(tail_hbm, tail_v)
        pltpu.sync_copy(tail_v, lin_hbm.at[pl.ds(TAIL0 * DEPTH, (VOCAB_ROWS - TAIL0) * DEPTH)])

    def stage(i, b):
        vb0 = pl.multiple_of((wid + 32 * i) * 128, 128)
        for a in range(AT):
            pltpu.async_copy(
                tiles_hbm.at[pl.ds(a * 8, 8), pl.ds(vb0, 128)],
                tiles_bufs[b].at[pl.ds(a * 8, 8)],
                tsem,
            )

    def wait_tiles(b):
        for a in range(AT):
            pltpu.make_async_copy(
                tiles_hbm.at[pl.ds(0, 8), pl.ds(0, 128)],
                tiles_bufs[b].at[pl.ds(a * 8, 8)],
                tsem,
            ).wait()

    def transpose(b):
        @plsc.parallel_loop(0, 32, unroll=2)
        def kc_body(kc):
            dvec = dtab_v[pl.ds(kc * 16, 16)]
            st = sttab_v[pl.ds(kc * 16, 16)]
            for ub in range(8):
                vals = plsc.load_gather(tiles_bufs[b], [dvec, iota16 + ub * 16])
                plsc.store_scatter(rows_bufs[b], [st + ub * 16 * DEPTH], vals)

    def writeback(i, b):
        vb = wid + 32 * i
        base = pl.multiple_of(vb * (128 * DEPTH), 128 * DEPTH)
        pltpu.async_copy(
            rows_bufs[b],
            lin_hbm.at[pl.ds(base, 128 * DEPTH)],
            wsem,
        )

    def wait_writeback(b):
        pltpu.make_async_copy(
            lin_hbm.at[pl.ds(0, 128 * DEPTH)], rows_bufs[b], wsem
        ).wait()

    stage(0, 0)
    stage(1, 1)

    def pair_body(i, carry):
        for b in (0, 1):
            idx = 2 * i + b
            wait_tiles(b)

            @pl.when(idx >= 2)
            def _reclaim():
                wait_writeback(b)

            transpose(b)
            writeback(idx, b)

            @pl.when(idx + 2 < nvb)
            def _next():
                stage(idx + 2, b)

        return carry

    lax.fori_loop(0, 122, pair_body, 0)

    @pl.when(nvb > 244)
    def _tail():
        wait_tiles(0)
        wait_writeback(0)
        transpose(0)
        writeback(244, 0)

    wait_writeback(0)
    wait_writeback(1)


@functools.partial(
    pl.kernel,
    out_type=jax.ShapeDtypeStruct((OUT_FLAT,), jnp.float32),
    mesh=_mesh,
    compiler_params=pltpu.CompilerParams(
        use_tc_tiling_on_sc=False, needs_layout_passes=False
    ),
    scratch_types=[
        pltpu.VMEM((2, TG, 128), jnp.int32),
        pltpu.VMEM((2, GROUP_ROWS, DEPTH), jnp.float32),
        pltpu.VMEM((2, TRANS), jnp.float32),
        pltpu.VMEM((MAX_LEN * DEPTH,), jnp.float32),
        pltpu.VMEM((32 * 16,), jnp.int32),
        pltpu.VMEM((32 * 16,), jnp.int32),
        pltpu.SemaphoreType.DMA,
        pltpu.SemaphoreType.DMA,
        pltpu.SemaphoreType.DMA,
    ],
)
def _embed(seq_hbm, tok_hbm, pos_hbm, out_hbm,
           idx_v, rows_v, trans_v, pos_v, perm_v, skew_v,
           isem, gsem, wsem):
    wid = lax.axis_index("s") * NC + lax.axis_index("c")
    g0 = wid * GPW

    pltpu.sync_copy(pos_hbm, pos_v)
    iota16 = lax.iota(jnp.int32, 16)

    for kc in range(32):
        k = kc & 15
        c0 = (kc >> 4) * 16
        dvec = c0 + ((iota16 + k) & 15)
        fvec = ((dvec >> 3) << 12) + ((dvec & 7) << 7)
        perm_v[pl.ds(kc * 16, 16)] = dvec
        skew_v[pl.ds(kc * 16, 16)] = fvec + iota16

    def coords(g):
        l = g // (TT // TG)
        tg = g % (TT // TG)
        return l, l // 8, l % 8, tg * TG

    def stage_ids(g, b):
        _, p, s, t0 = coords(g)
        return [
            pltpu.async_copy(seq_hbm.at[p, pl.ds(t0, TG), s], idx_v.at[b], isem)
        ]

    def launch_gathers(b):
        for tl in range(TG):
            pltpu.async_copy(
                tok_hbm.at[idx_v.at[b, tl]],
                rows_v.at[b].at[pl.ds(tl * 128, 128)],
                gsem,
            )

    def wait_gathers(b):
        pltpu.make_async_copy(
            tok_hbm.at[pl.ds(0, GROUP_ROWS)], rows_v.at[b], gsem
        ).wait()

    def transpose_add(g, b):
        l = g // (TT // TG)
        lbase = l * DEPTH

        @plsc.parallel_loop(0, 32, unroll=2)
        def kc_body(kc):
            dvec = perm_v[pl.ds(kc * 16, 16)]
            skew = skew_v[pl.ds(kc * 16, 16)]
            ps = plsc.load_gather(pos_v, [lbase + dvec])
            for tl in range(TG):
                for rb in range(8):
                    r0 = tl * 128 + rb * 16
                    vals = plsc.load_gather(rows_v.at[b], [iota16 + r0, dvec])
                    plsc.store_scatter(
                        trans_v.at[b],
                        [skew + (tl * 1024 + rb * 16)],
                        vals + ps,
                    )

    def writeback(g, b):
        l, _, _, t0 = coords(g)
        base = pl.multiple_of(l * (AT * TT * 1024) + t0 * 1024, 1024)
        return [
            pltpu.async_copy(
                trans_v.at[b].at[pl.ds(a * 4096, 4096)],
                out_hbm.at[pl.ds(base + a * (TT * 1024), 4096)],
                wsem,
            )
            for a in range(AT)
        ]

    for b in (0, 1):
        for c in stage_ids(g0 + b, b):
            c.wait()
        launch_gathers(b)

    def wait_writeback(b):
        pltpu.make_async_copy(
            out_hbm.at[pl.ds(0, TRANS)], trans_v.at[b], wsem
        ).wait()

    def pair_body(i, carry):
        for b in (0, 1):
            ci = 2 * i + b
            g = g0 + ci
            wait_gathers(b)

            @pl.when(ci >= 2)
            def _reclaim():
                wait_writeback(b)

            transpose_add(g, b)
            writeback(g, b)

            @pl.when(ci + 2 < GPW)
            def _next():
                ids = stage_ids(g + 2, b)
                for c in ids:
                    c.wait()
                launch_gathers(b)

        return carry

    lax.fori_loop(0, GPW // 2, pair_body, 0)
    wait_writeback(0)
    wait_writeback(1)


def kernel(seq, token_table, pos_table):
    seq4d = seq.T.reshape(PT, 8, TT, 128).transpose(0, 2, 1, 3)
    tok_tail = token_table[TAIL0:].reshape(-1)
    tok_lin = _table_transpose(token_table.T, tok_tail).reshape(
        VOCAB_ROWS, DEPTH
    )
    out_flat = _embed(seq4d, tok_lin, pos_table.reshape(-1))
    out5d = out_flat.reshape(MAX_LEN, AT, TT, 8, 128)
    return out5d.transpose(2, 4, 0, 1, 3).reshape(BATCH, MAX_LEN, DEPTH)
